# Initial kernel scaffold; baseline (speedup 1.0000x reference)
import jax
import jax.numpy as jnp
from jax.experimental import pallas as pl


def kernel(x, edge_index, edge_weight, batch, params):
    raise NotImplementedError("write your pallas kernel here")


# speedup vs baseline: 3.4037x; 3.4037x over previous
import functools

import jax
import jax.numpy as jnp
from jax import lax
from jax.experimental import pallas as pl
from jax.experimental.pallas import tpu as pltpu
from jax.experimental.pallas import tpu_sc as plsc

NN = 10000
NPAD = 10240
EE = 160000
EPAD = 163840
NG = 32
BN = 1024
NBLK = NPAD // BN

_f32 = jnp.float32


def _sc_prop_body(u_hbm, src_hbm, dst_hbm, v_hbm, src_v, dst_v, gbuf, slab,
                  sem):
    c = lax.axis_index("c")
    s = lax.axis_index("s")

    pltpu.sync_copy(src_hbm.at[s], src_v)
    pltpu.sync_copy(dst_hbm.at[s], dst_v)

    zero16 = jnp.zeros((16,), _f32)

    def zrow(r, carry):
        for k in range(8):
            gbuf[r, pl.ds(k * 16, 16)] = zero16
        return carry

    lax.fori_loop(0, 128, zrow, 0)
    base = s * 640
    for r in range(5):
        pltpu.sync_copy(gbuf, slab.at[pl.ds(base + r * 128, 128)])
    plsc.subcore_barrier()

    for cc in range(2):
        @pl.when(c == cc)
        def _():
            u_plane = u_hbm.at[cc]
            v_plane = v_hbm.at[cc]

            def step(j, carry):
                pltpu.async_copy(u_plane.at[src_v.at[j]], gbuf, sem).wait()
                pltpu.sync_copy(gbuf, slab.at[dst_v.at[j]], add=True)
                return carry

            lax.fori_loop(0, 80, step, 0)
            plsc.subcore_barrier()
            pltpu.sync_copy(slab.at[pl.ds(base, 640)],
                            v_plane.at[pl.ds(base, 640)])


def _sc_prop(u, src_t, dst_t):
    mesh = plsc.VectorSubcoreMesh(core_axis_name="c", subcore_axis_name="s",
                                  num_cores=2, num_subcores=16)
    return pl.kernel(
        _sc_prop_body,
        out_type=jax.ShapeDtypeStruct((2, NPAD, 128), _f32),
        mesh=mesh,
        scratch_types=[
            pltpu.VMEM((80, 128), jnp.int32),
            pltpu.VMEM((80, 128), jnp.int32),
            pltpu.VMEM((128, 128), _f32),
            pltpu.VMEM_SHARED((NPAD, 128), _f32),
            pltpu.SemaphoreType.DMA,
        ],
    )(u, src_t, dst_t)


def _row_mask(n, bn):
    rows = n * bn + lax.broadcasted_iota(jnp.int32, (bn, 128), 0)
    return jnp.where(rows < NN, 1.0, 0.0).astype(_f32)


def _acc_stats(stats_ref, z, n, bn):
    m = _row_mask(n, bn)
    zm = z * m
    s1 = jnp.sum(zm, axis=0, keepdims=True)
    s2 = jnp.sum(zm * z, axis=0, keepdims=True)
    blk = jnp.concatenate([s1, s2, jnp.zeros((6, z.shape[1]), _f32)], axis=0)

    @pl.when(n == 0)
    def _():
        stats_ref[...] = blk

    @pl.when(n != 0)
    def _():
        stats_ref[...] = stats_ref[...] + blk


def _dot(a, b):
    return jax.lax.dot_general(a, b, (((1,), (0,)), ((), ())),
                               preferred_element_type=_f32)


def _dis_body(deg_ref, d_ref):
    deg = deg_ref[...]
    d_ref[...] = jnp.where(deg > 0.0, jax.lax.rsqrt(jnp.maximum(deg, 1e-12)),
                           0.0)


def _dis_kernel(degplane):
    return pl.pallas_call(
        _dis_body,
        out_shape=jax.ShapeDtypeStruct((NPAD, 128), _f32),
        grid=(NBLK,),
        in_specs=[pl.BlockSpec((BN, 128), lambda n: (n, 0))],
        out_specs=pl.BlockSpec((BN, 128), lambda n: (n, 0)),
    )(degplane)


def _mm_negd_body(x_ref, w_ref, d_ref, o_ref):
    o = _dot(x_ref[...], w_ref[...])
    o_ref[0] = -d_ref[...] * o


def _mm_negd(x, w, d):
    return pl.pallas_call(
        _mm_negd_body,
        out_shape=jax.ShapeDtypeStruct((2, NPAD, 128), _f32),
        grid=(2, NBLK),
        in_specs=[
            pl.BlockSpec((BN, x.shape[1]), lambda c, n: (n, 0)),
            pl.BlockSpec((x.shape[1], 128), lambda c, n: (0, c)),
            pl.BlockSpec((BN, 128), lambda c, n: (n, 0)),
        ],
        out_specs=pl.BlockSpec((1, BN, 128), lambda c, n: (c, n, 0)),
    )(x, w, d)


def _a2_body(x_ref, w_ref, v_ref, d_ref, b_ref, z_ref, st_ref):
    c = pl.program_id(0)
    n = pl.program_id(1)
    z = _dot(x_ref[...], w_ref[...]) + d_ref[...] * v_ref[0] + b_ref[...]
    z_ref[...] = z
    _acc_stats(st_ref, z, n, BN)


def _conv_a2(x, w0, v, d, b):
    oc = w0.shape[1]
    return pl.pallas_call(
        _a2_body,
        out_shape=[jax.ShapeDtypeStruct((NPAD, oc), _f32),
                   jax.ShapeDtypeStruct((8, oc), _f32)],
        grid=(oc // 128, NBLK),
        in_specs=[
            pl.BlockSpec((BN, x.shape[1]), lambda c, n: (n, 0)),
            pl.BlockSpec((x.shape[1], 128), lambda c, n: (0, c)),
            pl.BlockSpec((1, BN, 128), lambda c, n: (c, n, 0)),
            pl.BlockSpec((BN, 128), lambda c, n: (n, 0)),
            pl.BlockSpec((1, 128), lambda c, n: (0, c)),
        ],
        out_specs=[pl.BlockSpec((BN, 128), lambda c, n: (n, c)),
                   pl.BlockSpec((8, 128), lambda c, n: (0, c))],
    )(x, w0, v, d, b)


def _b2_body(x_ref, v1_ref, v2_ref, d_ref, w0_ref, w1_ref, w2_ref, b_ref,
             z_ref, st_ref):
    n = pl.program_id(1)
    d = d_ref[...]
    t1 = jnp.concatenate([d * v1_ref[0], d * v1_ref[1]], axis=1)
    t2 = jnp.concatenate([d * v2_ref[0], d * v2_ref[1]], axis=1)
    z = (_dot(x_ref[...], w0_ref[...]) + _dot(t1, w1_ref[...]) +
         _dot(t2, w2_ref[...]) + b_ref[...])
    z_ref[...] = z
    _acc_stats(st_ref, z, n, BN)


def _conv_b2(x, v1, v2, d, w0m, w1, w2t, b):
    ic = x.shape[1]
    oc = w0m.shape[1]
    return pl.pallas_call(
        _b2_body,
        out_shape=[jax.ShapeDtypeStruct((NPAD, oc), _f32),
                   jax.ShapeDtypeStruct((8, oc), _f32)],
        grid=(oc // 128, NBLK),
        in_specs=[
            pl.BlockSpec((BN, ic), lambda c, n: (n, 0)),
            pl.BlockSpec((2, BN, 128), lambda c, n: (0, n, 0)),
            pl.BlockSpec((2, BN, 128), lambda c, n: (0, n, 0)),
            pl.BlockSpec((BN, 128), lambda c, n: (n, 0)),
            pl.BlockSpec((ic, 128), lambda c, n: (0, c)),
            pl.BlockSpec((ic, 128), lambda c, n: (0, c)),
            pl.BlockSpec((ic, 128), lambda c, n: (0, c)),
            pl.BlockSpec((1, 128), lambda c, n: (0, c)),
        ],
        out_specs=[pl.BlockSpec((BN, 128), lambda c, n: (n, c)),
                   pl.BlockSpec((8, 128), lambda c, n: (0, c))],
    )(x, v1, v2, d, w0m, w1, w2t, b)


def _mm_body(x_ref, w_ref, y_ref):
    y_ref[...] = _dot(x_ref[...], w_ref[...])


def _mm_plain(x, w):
    oc = w.shape[1]
    return pl.pallas_call(
        _mm_body,
        out_shape=jax.ShapeDtypeStruct((NPAD, oc), _f32),
        grid=(oc // 128, NBLK),
        in_specs=[
            pl.BlockSpec((BN, x.shape[1]), lambda c, n: (n, 0)),
            pl.BlockSpec((x.shape[1], 128), lambda c, n: (0, c)),
        ],
        out_specs=pl.BlockSpec((BN, 128), lambda c, n: (n, c)),
    )(x, w)


def _c1b_body(y_ref, d_ref, o_ref):
    o_ref[0] = -d_ref[...] * y_ref[...]


def _scale_negd_cols(y, d, col0):
    cb = col0 // 128
    return pl.pallas_call(
        _c1b_body,
        out_shape=jax.ShapeDtypeStruct((2, NPAD, 128), _f32),
        grid=(2, NBLK),
        in_specs=[
            pl.BlockSpec((BN, 128), lambda c, n: (n, c + cb)),
            pl.BlockSpec((BN, 128), lambda c, n: (n, 0)),
        ],
        out_specs=pl.BlockSpec((1, BN, 128), lambda c, n: (c, n, 0)),
    )(y, d)


def _b1_body(v_ref, d_ref, o_ref):
    d = d_ref[...]
    o_ref[0] = -d * d * v_ref[0]


def _scale_negd2(v, d):
    return pl.pallas_call(
        _b1_body,
        out_shape=jax.ShapeDtypeStruct((2, NPAD, 128), _f32),
        grid=(2, NBLK),
        in_specs=[
            pl.BlockSpec((1, BN, 128), lambda c, n: (c, n, 0)),
            pl.BlockSpec((BN, 128), lambda c, n: (n, 0)),
        ],
        out_specs=pl.BlockSpec((1, BN, 128), lambda c, n: (c, n, 0)),
    )(v, d)


def _c2_body(y_ref, v_ref, d_ref, o_ref):
    d = d_ref[...]
    o_ref[0] = -d * y_ref[...] - 2.0 * d * d * v_ref[0]


def _scale_c2(y, v, d, col0):
    cb = col0 // 128
    return pl.pallas_call(
        _c2_body,
        out_shape=jax.ShapeDtypeStruct((2, NPAD, 128), _f32),
        grid=(2, NBLK),
        in_specs=[
            pl.BlockSpec((BN, 128), lambda c, n: (n, c + cb)),
            pl.BlockSpec((1, BN, 128), lambda c, n: (c, n, 0)),
            pl.BlockSpec((BN, 128), lambda c, n: (n, 0)),
        ],
        out_specs=pl.BlockSpec((1, BN, 128), lambda c, n: (c, n, 0)),
    )(y, v, d)


def _c3_body(y0_ref, y2_ref, v_ref, d_ref, b_ref, z_ref, st_ref):
    n = pl.program_id(1)
    z = y0_ref[...] - y2_ref[...] + d_ref[...] * v_ref[0] + b_ref[...]
    z_ref[...] = z
    _acc_stats(st_ref, z, n, BN)


def _conv_c3(y, v, d, b):
    oc = 256
    return pl.pallas_call(
        _c3_body,
        out_shape=[jax.ShapeDtypeStruct((NPAD, oc), _f32),
                   jax.ShapeDtypeStruct((8, oc), _f32)],
        grid=(oc // 128, NBLK),
        in_specs=[
            pl.BlockSpec((BN, 128), lambda c, n: (n, c)),
            pl.BlockSpec((BN, 128), lambda c, n: (n, c + 4)),
            pl.BlockSpec((1, BN, 128), lambda c, n: (c, n, 0)),
            pl.BlockSpec((BN, 128), lambda c, n: (n, 0)),
            pl.BlockSpec((1, 128), lambda c, n: (0, c)),
        ],
        out_specs=[pl.BlockSpec((BN, 128), lambda c, n: (n, c)),
                   pl.BlockSpec((8, 128), lambda c, n: (0, c))],
    )(y, y, v, d, b)


def _d1_body(x_ref, v_ref, d_ref, w0_ref, w1_ref, b_ref, r_ref, h_ref):
    d = d_ref[...]
    t = jnp.concatenate([d * v_ref[0], d * v_ref[1]], axis=1)
    z = (_dot(x_ref[...], w0_ref[...]) + _dot(t, w1_ref[...]) + b_ref[...] +
         r_ref[...])
    h_ref[...] = jnp.maximum(z, 0.0)


def _conv_d1(x, v, d, w0, w1, b, res):
    ic = x.shape[1]
    oc = w0.shape[1]
    return pl.pallas_call(
        _d1_body,
        out_shape=jax.ShapeDtypeStruct((NPAD, oc), _f32),
        grid=(oc // 128, NBLK),
        in_specs=[
            pl.BlockSpec((BN, ic), lambda c, n: (n, 0)),
            pl.BlockSpec((2, BN, 128), lambda c, n: (0, n, 0)),
            pl.BlockSpec((BN, 128), lambda c, n: (n, 0)),
            pl.BlockSpec((ic, 128), lambda c, n: (0, c)),
            pl.BlockSpec((ic, 128), lambda c, n: (0, c)),
            pl.BlockSpec((1, 128), lambda c, n: (0, c)),
            pl.BlockSpec((BN, 128), lambda c, n: (n, c)),
        ],
        out_specs=pl.BlockSpec((BN, 128), lambda c, n: (n, c)),
    )(x, v, d, w0, w1, b, res)


def _norm_body_u(z_ref, st_ref, w_ref, b_ref, ms_ref, d_ref, h_ref, u_ref):
    _norm_core(z_ref, st_ref, w_ref, b_ref, ms_ref, h_ref, d_ref, u_ref)


def _norm_body(z_ref, st_ref, w_ref, b_ref, ms_ref, h_ref):
    _norm_core(z_ref, st_ref, w_ref, b_ref, ms_ref, h_ref, None, None)


def _norm_core(z_ref, st_ref, w_ref, b_ref, ms_ref, h_ref, d_ref, u_ref):
    z = z_ref[...]
    s1 = st_ref[0:1, :]
    s2 = st_ref[1:2, :]
    inv_n = 1.0 / float(NN)
    mean = s1 * inv_n
    mms = mean * ms_ref[...]
    var = s2 * inv_n - 2.0 * mms * mean + mms * mms
    out = z - mms
    h = w_ref[...] * out * jax.lax.rsqrt(var + 1e-5) + b_ref[...]
    h = jnp.where(h > 0.0, h, 0.2 * h)
    h_ref[...] = h
    if u_ref is not None:
        u_ref[0] = -d_ref[...] * h


def _norm_act(z, stats, w, b, ms):
    oc = z.shape[1]
    return pl.pallas_call(
        _norm_body,
        out_shape=jax.ShapeDtypeStruct((NPAD, oc), _f32),
        grid=(oc // 128, NBLK),
        in_specs=[
            pl.BlockSpec((BN, 128), lambda c, n: (n, c)),
            pl.BlockSpec((8, 128), lambda c, n: (0, c)),
            pl.BlockSpec((1, 128), lambda c, n: (0, c)),
            pl.BlockSpec((1, 128), lambda c, n: (0, c)),
            pl.BlockSpec((1, 128), lambda c, n: (0, c)),
        ],
        out_specs=pl.BlockSpec((BN, 128), lambda c, n: (n, c)),
    )(z, stats, w, b, ms)


def _norm_act_u(z, stats, w, b, ms, d):
    oc = z.shape[1]
    assert oc == 256
    return pl.pallas_call(
        _norm_body_u,
        out_shape=[jax.ShapeDtypeStruct((NPAD, oc), _f32),
                   jax.ShapeDtypeStruct((2, NPAD, 128), _f32)],
        grid=(oc // 128, NBLK),
        in_specs=[
            pl.BlockSpec((BN, 128), lambda c, n: (n, c)),
            pl.BlockSpec((8, 128), lambda c, n: (0, c)),
            pl.BlockSpec((1, 128), lambda c, n: (0, c)),
            pl.BlockSpec((1, 128), lambda c, n: (0, c)),
            pl.BlockSpec((1, 128), lambda c, n: (0, c)),
            pl.BlockSpec((BN, 128), lambda c, n: (n, 0)),
        ],
        out_specs=[pl.BlockSpec((BN, 128), lambda c, n: (n, c)),
                   pl.BlockSpec((1, BN, 128), lambda c, n: (c, n, 0))],
    )(z, stats, w, b, ms, d)


def _pool_body(h_ref, b_ref, s_ref, c_ref):
    n = pl.program_id(0)
    onehot = (b_ref[...] == lax.broadcasted_iota(jnp.int32, (1, NG), 1))
    onehot = onehot.astype(_f32)
    sums = jax.lax.dot_general(onehot, h_ref[...], (((0,), (0,)), ((), ())),
                               preferred_element_type=_f32)
    ones = jnp.ones((BN, 128), _f32)
    cnt = jax.lax.dot_general(onehot, ones, (((0,), (0,)), ((), ())),
                              preferred_element_type=_f32)

    @pl.when(n == 0)
    def _():
        s_ref[...] = sums
        c_ref[...] = cnt

    @pl.when(n != 0)
    def _():
        s_ref[...] = s_ref[...] + sums
        c_ref[...] = c_ref[...] + cnt


def _pool(h, batch2d):
    return pl.pallas_call(
        _pool_body,
        out_shape=[jax.ShapeDtypeStruct((NG, 512), _f32),
                   jax.ShapeDtypeStruct((NG, 128), _f32)],
        grid=(NBLK,),
        in_specs=[
            pl.BlockSpec((BN, 512), lambda n: (n, 0)),
            pl.BlockSpec((BN, 1), lambda n: (n, 0)),
        ],
        out_specs=[pl.BlockSpec((NG, 512), lambda n: (0, 0)),
                   pl.BlockSpec((NG, 128), lambda n: (0, 0))],
    )(h, batch2d)


def _head_body(s_ref, c_ref, w1_ref, b1_ref, w2_ref, b2_ref, o_ref):
    cnt = jnp.maximum(c_ref[:, 0:1], 1.0)
    pooled = s_ref[...] / cnt
    t = jnp.tanh(_dot(pooled, w1_ref[...]) + b1_ref[...])
    o_ref[...] = _dot(t, w2_ref[...]) + b2_ref[...]


def _head(sums, cnt, w1, b1, w2, b2):
    return pl.pallas_call(
        _head_body,
        out_shape=jax.ShapeDtypeStruct((NG, 12), _f32),
    )(sums, cnt, w1, b1, w2, b2)


def _prep_edges(edge_index):
    src = edge_index[0]
    dst = edge_index[1]
    pad = EPAD - EE
    src_p = jnp.concatenate([src, jnp.full((pad,), NN, jnp.int32)])
    dst_p = jnp.concatenate([dst, jnp.full((pad,), NN, jnp.int32)])
    return src_p.reshape(16, 80, 128), dst_p.reshape(16, 80, 128)


def _planes(a):
    return jnp.moveaxis(a.reshape(NPAD, 2, 128), 1, 0)


@jax.jit
def kernel(x, edge_index, edge_weight, batch, params):
    del edge_weight
    src_t, dst_t = _prep_edges(edge_index)

    xp = jnp.pad(x, ((0, NPAD - NN), (0, 0)))
    batch_p = jnp.pad(batch, (0, NPAD - NN), constant_values=NG)
    batch2d = batch_p.reshape(NPAD, 1)

    ones_planes = jnp.ones((2, NPAD, 128), _f32)
    degp = _sc_prop(ones_planes, dst_t, src_t)
    d = _dis_kernel(degp[0])

    def bias2d(bv):
        return bv.reshape(1, -1)

    h = xp
    for blk in range(4):
        i0 = blk * 4
        pA = params['conv%d' % (i0 + 1)]
        pB = params['conv%d' % (i0 + 2)]
        pC = params['conv%d' % (i0 + 3)]
        pD = params['conv%d' % (i0 + 4)]
        gA = params['bn%d' % (i0 + 1)]
        gB = params['bn%d' % (i0 + 2)]
        gC = params['bn%d' % (i0 + 3)]

        u = _mm_negd(h, pA['Ws'][1], d)
        v = _sc_prop(u, src_t, dst_t)
        z, st = _conv_a2(h, pA['Ws'][0], v, d, bias2d(pA['b']))
        h1, u0 = _norm_act_u(z, st, bias2d(gA['weight']), bias2d(gA['bias']),
                             bias2d(gA['mean_scale']), d)

        v1 = _sc_prop(u0, src_t, dst_t)
        u1 = _scale_negd2(v1, d)
        v2 = _sc_prop(u1, src_t, dst_t)
        w0m = pB['Ws'][0] - pB['Ws'][2]
        w2t = 2.0 * pB['Ws'][2]
        z, st = _conv_b2(h1, v1, v2, d, w0m, pB['Ws'][1], w2t, bias2d(pB['b']))
        h2 = _norm_act(z, st, bias2d(gB['weight']), bias2d(gB['bias']),
                       bias2d(gB['mean_scale']))

        wcat = jnp.concatenate([pC['Ws'][0], pC['Ws'][1], pC['Ws'][2]], axis=1)
        y = _mm_plain(h2, wcat)
        u2 = _scale_negd_cols(y, d, 512)
        vA = _sc_prop(u2, src_t, dst_t)
        um = _scale_c2(y, vA, d, 256)
        vB = _sc_prop(um, src_t, dst_t)
        z, st = _conv_c3(y, vB, d, bias2d(pC['b']))
        h3, u3 = _norm_act_u(z, st, bias2d(gC['weight']), bias2d(gC['bias']),
                             bias2d(gC['mean_scale']), d)

        v = _sc_prop(u3, src_t, dst_t)
        h = _conv_d1(h3, v, d, pD['Ws'][0], pD['Ws'][1], bias2d(pD['b']), xp)

    sums, cnt = _pool(h, batch2d)
    return _head(sums, cnt, params['linear1']['W'],
                 bias2d(params['linear1']['b']), params['linear2']['W'],
                 bias2d(params['linear2']['b']))


# speedup vs baseline: 3.7828x; 1.1114x over previous
import functools

import jax
import jax.numpy as jnp
from jax import lax
from jax.experimental import pallas as pl
from jax.experimental.pallas import tpu as pltpu
from jax.experimental.pallas import tpu_sc as plsc

NN = 10000
NPAD = 10240
EE = 160000
EPAD = 163840
NG = 32
BN = 1024
NBLK = NPAD // BN

_f32 = jnp.float32


def _sc_prop_body(u_hbm, src_hbm, dst_hbm, v_hbm, src_v, dst_v, gbuf, slab,
                  g0, g1, s0, s1, is0, is1, id0, id1):
    gsems = [g0, g1]
    ssems = [s0, s1]
    isems = [is0, is1]
    idsems = [id0, id1]
    c = lax.axis_index("c")
    s = lax.axis_index("s")

    zero16 = jnp.zeros((16,), _f32)

    def zrow(r, carry):
        for k in range(8):
            gbuf[0, r, pl.ds(k * 16, 16)] = zero16
        return carry

    lax.fori_loop(0, 128, zrow, 0)
    base = s * 640
    for r in range(5):
        pltpu.sync_copy(gbuf.at[0], slab.at[pl.ds(base + r * 128, 128)])
    plsc.subcore_barrier()

    def load_idx_group_sync(G, slot):
        pltpu.sync_copy(src_hbm.at[s, pl.ds(4 * G, 4)], src_v.at[slot])
        pltpu.sync_copy(dst_hbm.at[s, pl.ds(4 * G, 4)], dst_v.at[slot])

    load_idx_group_sync(0, 0)
    load_idx_group_sync(1, 1)

    for cc in range(2):
        @pl.when(c == cc)
        def _():
            u_plane = u_hbm.at[cc]
            v_plane = v_hbm.at[cc]

            def start_gather(idx_ref, b):
                pltpu.async_copy(u_plane.at[idx_ref], gbuf.at[b], gsems[b])

            def wait_gather(b):
                pltpu.make_async_copy(u_plane.at[src_v.at[0, 0]], gbuf.at[b],
                                      gsems[b]).wait()

            def start_scatter(idx_ref, b):
                pltpu.async_copy(gbuf.at[b], slab.at[idx_ref], ssems[b],
                                 add=True)

            def wait_scatter(b):
                pltpu.make_async_copy(gbuf.at[b], slab.at[dst_v.at[0, 0]],
                                      ssems[b]).wait()

            start_gather(src_v.at[0, 0], 0)

            def outer(g, carry):
                gg = lax.rem(g, 2)
                gn = lax.rem(g + 1, 2)
                for b in range(4):
                    j4 = 4 * g + b
                    sb = b % 2
                    ob = 1 - sb

                    wait_gather(sb)
                    start_scatter(dst_v.at[gg, b], sb)

                    @pl.when(j4 >= 1)
                    def _():
                        wait_scatter(ob)

                    if b == 0:
                        @pl.when(jnp.logical_and(g >= 1, g <= 18))
                        def _():
                            pltpu.async_copy(
                                src_hbm.at[s, pl.ds(4 * (g + 1), 4)],
                                src_v.at[gn], isems[0])
                            pltpu.async_copy(
                                dst_hbm.at[s, pl.ds(4 * (g + 1), 4)],
                                dst_v.at[gn], idsems[0])

                    if b < 3:
                        start_gather(src_v.at[gg, b + 1], ob)
                    else:
                        @pl.when(g < 19)
                        def _():
                            @pl.when(g >= 1)
                            def _():
                                pltpu.make_async_copy(
                                    src_hbm.at[s, pl.ds(0, 4)],
                                    src_v.at[gn], isems[0]).wait()
                                pltpu.make_async_copy(
                                    dst_hbm.at[s, pl.ds(0, 4)],
                                    dst_v.at[gn], idsems[0]).wait()
                            start_gather(src_v.at[gn, 0], ob)
                return carry

            lax.fori_loop(0, 20, outer, 0)
            wait_scatter(1)
            plsc.subcore_barrier()
            pltpu.sync_copy(slab.at[pl.ds(base, 640)],
                            v_plane.at[pl.ds(base, 640)])


def _sc_prop(u, src_t, dst_t):
    mesh = plsc.VectorSubcoreMesh(core_axis_name="c", subcore_axis_name="s",
                                  num_cores=2, num_subcores=16)
    return pl.kernel(
        _sc_prop_body,
        out_type=jax.ShapeDtypeStruct((2, NPAD, 128), _f32),
        mesh=mesh,
        scratch_types=[
            pltpu.VMEM((2, 4, 128), jnp.int32),
            pltpu.VMEM((2, 4, 128), jnp.int32),
            pltpu.VMEM((2, 128, 128), _f32),
            pltpu.VMEM_SHARED((NPAD, 128), _f32),
        ] + [pltpu.SemaphoreType.DMA] * 8,
    )(u, src_t, dst_t)


def _row_mask(n, bn):
    rows = n * bn + lax.broadcasted_iota(jnp.int32, (bn, 128), 0)
    return jnp.where(rows < NN, 1.0, 0.0).astype(_f32)


def _acc_stats(stats_ref, z, n, bn):
    m = _row_mask(n, bn)
    zm = z * m
    s1 = jnp.sum(zm, axis=0, keepdims=True)
    s2 = jnp.sum(zm * z, axis=0, keepdims=True)
    blk = jnp.concatenate([s1, s2, jnp.zeros((6, z.shape[1]), _f32)], axis=0)

    @pl.when(n == 0)
    def _():
        stats_ref[...] = blk

    @pl.when(n != 0)
    def _():
        stats_ref[...] = stats_ref[...] + blk


def _dot(a, b):
    return jax.lax.dot_general(a, b, (((1,), (0,)), ((), ())),
                               preferred_element_type=_f32)


def _dis_body(deg_ref, d_ref):
    deg = deg_ref[...]
    d_ref[...] = jnp.where(deg > 0.0, jax.lax.rsqrt(jnp.maximum(deg, 1e-12)),
                           0.0)


def _dis_kernel(degplane):
    return pl.pallas_call(
        _dis_body,
        out_shape=jax.ShapeDtypeStruct((NPAD, 128), _f32),
        grid=(NBLK,),
        in_specs=[pl.BlockSpec((BN, 128), lambda n: (n, 0))],
        out_specs=pl.BlockSpec((BN, 128), lambda n: (n, 0)),
    )(degplane)


def _mm_negd_body(x_ref, w_ref, d_ref, o_ref):
    o = _dot(x_ref[...], w_ref[...])
    o_ref[0] = -d_ref[...] * o


def _mm_negd(x, w, d):
    return pl.pallas_call(
        _mm_negd_body,
        out_shape=jax.ShapeDtypeStruct((2, NPAD, 128), _f32),
        grid=(2, NBLK),
        in_specs=[
            pl.BlockSpec((BN, x.shape[1]), lambda c, n: (n, 0)),
            pl.BlockSpec((x.shape[1], 128), lambda c, n: (0, c)),
            pl.BlockSpec((BN, 128), lambda c, n: (n, 0)),
        ],
        out_specs=pl.BlockSpec((1, BN, 128), lambda c, n: (c, n, 0)),
    )(x, w, d)


def _a2_body(x_ref, w_ref, v_ref, d_ref, b_ref, z_ref, st_ref):
    c = pl.program_id(0)
    n = pl.program_id(1)
    z = _dot(x_ref[...], w_ref[...]) + d_ref[...] * v_ref[0] + b_ref[...]
    z_ref[...] = z
    _acc_stats(st_ref, z, n, BN)


def _conv_a2(x, w0, v, d, b):
    oc = w0.shape[1]
    return pl.pallas_call(
        _a2_body,
        out_shape=[jax.ShapeDtypeStruct((NPAD, oc), _f32),
                   jax.ShapeDtypeStruct((8, oc), _f32)],
        grid=(oc // 128, NBLK),
        in_specs=[
            pl.BlockSpec((BN, x.shape[1]), lambda c, n: (n, 0)),
            pl.BlockSpec((x.shape[1], 128), lambda c, n: (0, c)),
            pl.BlockSpec((1, BN, 128), lambda c, n: (c, n, 0)),
            pl.BlockSpec((BN, 128), lambda c, n: (n, 0)),
            pl.BlockSpec((1, 128), lambda c, n: (0, c)),
        ],
        out_specs=[pl.BlockSpec((BN, 128), lambda c, n: (n, c)),
                   pl.BlockSpec((8, 128), lambda c, n: (0, c))],
    )(x, w0, v, d, b)


def _b2_body(x_ref, v1_ref, v2_ref, d_ref, w0_ref, w1_ref, w2_ref, b_ref,
             z_ref, st_ref):
    n = pl.program_id(1)
    d = d_ref[...]
    t1 = jnp.concatenate([d * v1_ref[0], d * v1_ref[1]], axis=1)
    t2 = jnp.concatenate([d * v2_ref[0], d * v2_ref[1]], axis=1)
    z = (_dot(x_ref[...], w0_ref[...]) + _dot(t1, w1_ref[...]) +
         _dot(t2, w2_ref[...]) + b_ref[...])
    z_ref[...] = z
    _acc_stats(st_ref, z, n, BN)


def _conv_b2(x, v1, v2, d, w0m, w1, w2t, b):
    ic = x.shape[1]
    oc = w0m.shape[1]
    return pl.pallas_call(
        _b2_body,
        out_shape=[jax.ShapeDtypeStruct((NPAD, oc), _f32),
                   jax.ShapeDtypeStruct((8, oc), _f32)],
        grid=(oc // 128, NBLK),
        in_specs=[
            pl.BlockSpec((BN, ic), lambda c, n: (n, 0)),
            pl.BlockSpec((2, BN, 128), lambda c, n: (0, n, 0)),
            pl.BlockSpec((2, BN, 128), lambda c, n: (0, n, 0)),
            pl.BlockSpec((BN, 128), lambda c, n: (n, 0)),
            pl.BlockSpec((ic, 128), lambda c, n: (0, c)),
            pl.BlockSpec((ic, 128), lambda c, n: (0, c)),
            pl.BlockSpec((ic, 128), lambda c, n: (0, c)),
            pl.BlockSpec((1, 128), lambda c, n: (0, c)),
        ],
        out_specs=[pl.BlockSpec((BN, 128), lambda c, n: (n, c)),
                   pl.BlockSpec((8, 128), lambda c, n: (0, c))],
    )(x, v1, v2, d, w0m, w1, w2t, b)


def _mm_body(x_ref, w_ref, y_ref):
    y_ref[...] = _dot(x_ref[...], w_ref[...])


def _mm_plain(x, w):
    oc = w.shape[1]
    return pl.pallas_call(
        _mm_body,
        out_shape=jax.ShapeDtypeStruct((NPAD, oc), _f32),
        grid=(oc // 128, NBLK),
        in_specs=[
            pl.BlockSpec((BN, x.shape[1]), lambda c, n: (n, 0)),
            pl.BlockSpec((x.shape[1], 128), lambda c, n: (0, c)),
        ],
        out_specs=pl.BlockSpec((BN, 128), lambda c, n: (n, c)),
    )(x, w)


def _c1b_body(y_ref, d_ref, o_ref):
    o_ref[0] = -d_ref[...] * y_ref[...]


def _scale_negd_cols(y, d, col0):
    cb = col0 // 128
    return pl.pallas_call(
        _c1b_body,
        out_shape=jax.ShapeDtypeStruct((2, NPAD, 128), _f32),
        grid=(2, NBLK),
        in_specs=[
            pl.BlockSpec((BN, 128), lambda c, n: (n, c + cb)),
            pl.BlockSpec((BN, 128), lambda c, n: (n, 0)),
        ],
        out_specs=pl.BlockSpec((1, BN, 128), lambda c, n: (c, n, 0)),
    )(y, d)


def _b1_body(v_ref, d_ref, o_ref):
    d = d_ref[...]
    o_ref[0] = -d * d * v_ref[0]


def _scale_negd2(v, d):
    return pl.pallas_call(
        _b1_body,
        out_shape=jax.ShapeDtypeStruct((2, NPAD, 128), _f32),
        grid=(2, NBLK),
        in_specs=[
            pl.BlockSpec((1, BN, 128), lambda c, n: (c, n, 0)),
            pl.BlockSpec((BN, 128), lambda c, n: (n, 0)),
        ],
        out_specs=pl.BlockSpec((1, BN, 128), lambda c, n: (c, n, 0)),
    )(v, d)


def _c2_body(y_ref, v_ref, d_ref, o_ref):
    d = d_ref[...]
    o_ref[0] = -d * y_ref[...] - 2.0 * d * d * v_ref[0]


def _scale_c2(y, v, d, col0):
    cb = col0 // 128
    return pl.pallas_call(
        _c2_body,
        out_shape=jax.ShapeDtypeStruct((2, NPAD, 128), _f32),
        grid=(2, NBLK),
        in_specs=[
            pl.BlockSpec((BN, 128), lambda c, n: (n, c + cb)),
            pl.BlockSpec((1, BN, 128), lambda c, n: (c, n, 0)),
            pl.BlockSpec((BN, 128), lambda c, n: (n, 0)),
        ],
        out_specs=pl.BlockSpec((1, BN, 128), lambda c, n: (c, n, 0)),
    )(y, v, d)


def _c3_body(y0_ref, y2_ref, v_ref, d_ref, b_ref, z_ref, st_ref):
    n = pl.program_id(1)
    z = y0_ref[...] - y2_ref[...] + d_ref[...] * v_ref[0] + b_ref[...]
    z_ref[...] = z
    _acc_stats(st_ref, z, n, BN)


def _conv_c3(y, v, d, b):
    oc = 256
    return pl.pallas_call(
        _c3_body,
        out_shape=[jax.ShapeDtypeStruct((NPAD, oc), _f32),
                   jax.ShapeDtypeStruct((8, oc), _f32)],
        grid=(oc // 128, NBLK),
        in_specs=[
            pl.BlockSpec((BN, 128), lambda c, n: (n, c)),
            pl.BlockSpec((BN, 128), lambda c, n: (n, c + 4)),
            pl.BlockSpec((1, BN, 128), lambda c, n: (c, n, 0)),
            pl.BlockSpec((BN, 128), lambda c, n: (n, 0)),
            pl.BlockSpec((1, 128), lambda c, n: (0, c)),
        ],
        out_specs=[pl.BlockSpec((BN, 128), lambda c, n: (n, c)),
                   pl.BlockSpec((8, 128), lambda c, n: (0, c))],
    )(y, y, v, d, b)


def _d1_body(x_ref, v_ref, d_ref, w0_ref, w1_ref, b_ref, r_ref, h_ref):
    d = d_ref[...]
    t = jnp.concatenate([d * v_ref[0], d * v_ref[1]], axis=1)
    z = (_dot(x_ref[...], w0_ref[...]) + _dot(t, w1_ref[...]) + b_ref[...] +
         r_ref[...])
    h_ref[...] = jnp.maximum(z, 0.0)


def _conv_d1(x, v, d, w0, w1, b, res):
    ic = x.shape[1]
    oc = w0.shape[1]
    return pl.pallas_call(
        _d1_body,
        out_shape=jax.ShapeDtypeStruct((NPAD, oc), _f32),
        grid=(oc // 128, NBLK),
        in_specs=[
            pl.BlockSpec((BN, ic), lambda c, n: (n, 0)),
            pl.BlockSpec((2, BN, 128), lambda c, n: (0, n, 0)),
            pl.BlockSpec((BN, 128), lambda c, n: (n, 0)),
            pl.BlockSpec((ic, 128), lambda c, n: (0, c)),
            pl.BlockSpec((ic, 128), lambda c, n: (0, c)),
            pl.BlockSpec((1, 128), lambda c, n: (0, c)),
            pl.BlockSpec((BN, 128), lambda c, n: (n, c)),
        ],
        out_specs=pl.BlockSpec((BN, 128), lambda c, n: (n, c)),
    )(x, v, d, w0, w1, b, res)


def _norm_body_u(z_ref, st_ref, w_ref, b_ref, ms_ref, d_ref, h_ref, u_ref):
    _norm_core(z_ref, st_ref, w_ref, b_ref, ms_ref, h_ref, d_ref, u_ref)


def _norm_body(z_ref, st_ref, w_ref, b_ref, ms_ref, h_ref):
    _norm_core(z_ref, st_ref, w_ref, b_ref, ms_ref, h_ref, None, None)


def _norm_core(z_ref, st_ref, w_ref, b_ref, ms_ref, h_ref, d_ref, u_ref):
    z = z_ref[...]
    s1 = st_ref[0:1, :]
    s2 = st_ref[1:2, :]
    inv_n = 1.0 / float(NN)
    mean = s1 * inv_n
    mms = mean * ms_ref[...]
    var = s2 * inv_n - 2.0 * mms * mean + mms * mms
    out = z - mms
    h = w_ref[...] * out * jax.lax.rsqrt(var + 1e-5) + b_ref[...]
    h = jnp.where(h > 0.0, h, 0.2 * h)
    h_ref[...] = h
    if u_ref is not None:
        u_ref[0] = -d_ref[...] * h


def _norm_act(z, stats, w, b, ms):
    oc = z.shape[1]
    return pl.pallas_call(
        _norm_body,
        out_shape=jax.ShapeDtypeStruct((NPAD, oc), _f32),
        grid=(oc // 128, NBLK),
        in_specs=[
            pl.BlockSpec((BN, 128), lambda c, n: (n, c)),
            pl.BlockSpec((8, 128), lambda c, n: (0, c)),
            pl.BlockSpec((1, 128), lambda c, n: (0, c)),
            pl.BlockSpec((1, 128), lambda c, n: (0, c)),
            pl.BlockSpec((1, 128), lambda c, n: (0, c)),
        ],
        out_specs=pl.BlockSpec((BN, 128), lambda c, n: (n, c)),
    )(z, stats, w, b, ms)


def _norm_act_u(z, stats, w, b, ms, d):
    oc = z.shape[1]
    assert oc == 256
    return pl.pallas_call(
        _norm_body_u,
        out_shape=[jax.ShapeDtypeStruct((NPAD, oc), _f32),
                   jax.ShapeDtypeStruct((2, NPAD, 128), _f32)],
        grid=(oc // 128, NBLK),
        in_specs=[
            pl.BlockSpec((BN, 128), lambda c, n: (n, c)),
            pl.BlockSpec((8, 128), lambda c, n: (0, c)),
            pl.BlockSpec((1, 128), lambda c, n: (0, c)),
            pl.BlockSpec((1, 128), lambda c, n: (0, c)),
            pl.BlockSpec((1, 128), lambda c, n: (0, c)),
            pl.BlockSpec((BN, 128), lambda c, n: (n, 0)),
        ],
        out_specs=[pl.BlockSpec((BN, 128), lambda c, n: (n, c)),
                   pl.BlockSpec((1, BN, 128), lambda c, n: (c, n, 0))],
    )(z, stats, w, b, ms, d)


def _pool_body(h_ref, b_ref, s_ref, c_ref):
    n = pl.program_id(0)
    onehot = (b_ref[...] == lax.broadcasted_iota(jnp.int32, (1, NG), 1))
    onehot = onehot.astype(_f32)
    sums = jax.lax.dot_general(onehot, h_ref[...], (((0,), (0,)), ((), ())),
                               preferred_element_type=_f32)
    ones = jnp.ones((BN, 128), _f32)
    cnt = jax.lax.dot_general(onehot, ones, (((0,), (0,)), ((), ())),
                              preferred_element_type=_f32)

    @pl.when(n == 0)
    def _():
        s_ref[...] = sums
        c_ref[...] = cnt

    @pl.when(n != 0)
    def _():
        s_ref[...] = s_ref[...] + sums
        c_ref[...] = c_ref[...] + cnt


def _pool(h, batch2d):
    return pl.pallas_call(
        _pool_body,
        out_shape=[jax.ShapeDtypeStruct((NG, 512), _f32),
                   jax.ShapeDtypeStruct((NG, 128), _f32)],
        grid=(NBLK,),
        in_specs=[
            pl.BlockSpec((BN, 512), lambda n: (n, 0)),
            pl.BlockSpec((BN, 1), lambda n: (n, 0)),
        ],
        out_specs=[pl.BlockSpec((NG, 512), lambda n: (0, 0)),
                   pl.BlockSpec((NG, 128), lambda n: (0, 0))],
    )(h, batch2d)


def _head_body(s_ref, c_ref, w1_ref, b1_ref, w2_ref, b2_ref, o_ref):
    cnt = jnp.maximum(c_ref[:, 0:1], 1.0)
    pooled = s_ref[...] / cnt
    t = jnp.tanh(_dot(pooled, w1_ref[...]) + b1_ref[...])
    o_ref[...] = _dot(t, w2_ref[...]) + b2_ref[...]


def _head(sums, cnt, w1, b1, w2, b2):
    return pl.pallas_call(
        _head_body,
        out_shape=jax.ShapeDtypeStruct((NG, 12), _f32),
    )(sums, cnt, w1, b1, w2, b2)


def _prep_edges(edge_index):
    src = edge_index[0]
    dst = edge_index[1]
    pad = EPAD - EE
    src_p = jnp.concatenate([src, jnp.full((pad,), NN, jnp.int32)])
    dst_p = jnp.concatenate([dst, jnp.full((pad,), NN, jnp.int32)])
    return src_p.reshape(16, 80, 128), dst_p.reshape(16, 80, 128)


def _planes(a):
    return jnp.moveaxis(a.reshape(NPAD, 2, 128), 1, 0)


@jax.jit
def kernel(x, edge_index, edge_weight, batch, params):
    del edge_weight
    src_t, dst_t = _prep_edges(edge_index)

    xp = jnp.pad(x, ((0, NPAD - NN), (0, 0)))
    batch_p = jnp.pad(batch, (0, NPAD - NN), constant_values=NG)
    batch2d = batch_p.reshape(NPAD, 1)

    ones_planes = jnp.ones((2, NPAD, 128), _f32)
    degp = _sc_prop(ones_planes, dst_t, src_t)
    d = _dis_kernel(degp[0])

    def bias2d(bv):
        return bv.reshape(1, -1)

    h = xp
    for blk in range(4):
        i0 = blk * 4
        pA = params['conv%d' % (i0 + 1)]
        pB = params['conv%d' % (i0 + 2)]
        pC = params['conv%d' % (i0 + 3)]
        pD = params['conv%d' % (i0 + 4)]
        gA = params['bn%d' % (i0 + 1)]
        gB = params['bn%d' % (i0 + 2)]
        gC = params['bn%d' % (i0 + 3)]

        u = _mm_negd(h, pA['Ws'][1], d)
        v = _sc_prop(u, src_t, dst_t)
        z, st = _conv_a2(h, pA['Ws'][0], v, d, bias2d(pA['b']))
        h1, u0 = _norm_act_u(z, st, bias2d(gA['weight']), bias2d(gA['bias']),
                             bias2d(gA['mean_scale']), d)

        v1 = _sc_prop(u0, src_t, dst_t)
        u1 = _scale_negd2(v1, d)
        v2 = _sc_prop(u1, src_t, dst_t)
        w0m = pB['Ws'][0] - pB['Ws'][2]
        w2t = 2.0 * pB['Ws'][2]
        z, st = _conv_b2(h1, v1, v2, d, w0m, pB['Ws'][1], w2t, bias2d(pB['b']))
        h2 = _norm_act(z, st, bias2d(gB['weight']), bias2d(gB['bias']),
                       bias2d(gB['mean_scale']))

        wcat = jnp.concatenate([pC['Ws'][0], pC['Ws'][1], pC['Ws'][2]], axis=1)
        y = _mm_plain(h2, wcat)
        u2 = _scale_negd_cols(y, d, 512)
        vA = _sc_prop(u2, src_t, dst_t)
        um = _scale_c2(y, vA, d, 256)
        vB = _sc_prop(um, src_t, dst_t)
        z, st = _conv_c3(y, vB, d, bias2d(pC['b']))
        h3, u3 = _norm_act_u(z, st, bias2d(gC['weight']), bias2d(gC['bias']),
                             bias2d(gC['mean_scale']), d)

        v = _sc_prop(u3, src_t, dst_t)
        h = _conv_d1(h3, v, d, pD['Ws'][0], pD['Ws'][1], bias2d(pD['b']), xp)

    sums, cnt = _pool(h, batch2d)
    return _head(sums, cnt, params['linear1']['W'],
                 bias2d(params['linear1']['b']), params['linear2']['W'],
                 bias2d(params['linear2']['b']))


# speedup vs baseline: 3.8295x; 1.0124x over previous
import functools

import jax
import jax.numpy as jnp
from jax import lax
from jax.experimental import pallas as pl
from jax.experimental.pallas import tpu as pltpu
from jax.experimental.pallas import tpu_sc as plsc

NN = 10000
NPAD = 10240
EE = 160000
EPAD = 163840
NG = 32
BN = 1024
NBLK = NPAD // BN

_f32 = jnp.float32


def _sc_prop_body(u_hbm, src_hbm, dst_hbm, v_hbm, src_v, dst_v, gbuf, slab,
                  g0, g1, s0, s1, is0, is1, id0, id1):
    gsems = [g0, g1]
    ssems = [s0, s1]
    isems = [is0, is1]
    idsems = [id0, id1]
    c = lax.axis_index("c")
    s = lax.axis_index("s")

    zero16 = jnp.zeros((16,), _f32)

    def zrow(r, carry):
        for k in range(8):
            gbuf[0, r, pl.ds(k * 16, 16)] = zero16
        return carry

    lax.fori_loop(0, 128, zrow, 0)
    base = s * 640
    for r in range(5):
        pltpu.sync_copy(gbuf.at[0], slab.at[pl.ds(base + r * 128, 128)])
    plsc.subcore_barrier()

    def load_idx_group_sync(G, slot):
        pltpu.sync_copy(src_hbm.at[s, pl.ds(4 * G, 4)], src_v.at[slot])
        pltpu.sync_copy(dst_hbm.at[s, pl.ds(4 * G, 4)], dst_v.at[slot])

    load_idx_group_sync(0, 0)
    load_idx_group_sync(1, 1)

    for cc in range(2):
        @pl.when(c == cc)
        def _():
            u_plane = u_hbm.at[cc]
            v_plane = v_hbm.at[cc]

            def start_gather(idx_ref, b):
                pltpu.async_copy(u_plane.at[idx_ref], gbuf.at[b], gsems[b])

            def wait_gather(b):
                pltpu.make_async_copy(u_plane.at[src_v.at[0, 0]], gbuf.at[b],
                                      gsems[b]).wait()

            def start_scatter(idx_ref, b):
                pltpu.async_copy(gbuf.at[b], slab.at[idx_ref], ssems[b],
                                 add=True)

            def wait_scatter(b):
                pltpu.make_async_copy(gbuf.at[b], slab.at[dst_v.at[0, 0]],
                                      ssems[b]).wait()

            start_gather(src_v.at[0, 0], 0)

            def outer(g, carry):
                gg = lax.rem(g, 2)
                gn = lax.rem(g + 1, 2)
                for b in range(4):
                    j4 = 4 * g + b
                    sb = b % 2
                    ob = 1 - sb

                    wait_gather(sb)
                    start_scatter(dst_v.at[gg, b], sb)

                    @pl.when(j4 >= 1)
                    def _():
                        wait_scatter(ob)

                    if b == 0:
                        @pl.when(jnp.logical_and(g >= 1, g <= 18))
                        def _():
                            pltpu.async_copy(
                                src_hbm.at[s, pl.ds(4 * (g + 1), 4)],
                                src_v.at[gn], isems[0])
                            pltpu.async_copy(
                                dst_hbm.at[s, pl.ds(4 * (g + 1), 4)],
                                dst_v.at[gn], idsems[0])

                    if b < 3:
                        start_gather(src_v.at[gg, b + 1], ob)
                    else:
                        @pl.when(g < 19)
                        def _():
                            @pl.when(g >= 1)
                            def _():
                                pltpu.make_async_copy(
                                    src_hbm.at[s, pl.ds(0, 4)],
                                    src_v.at[gn], isems[0]).wait()
                                pltpu.make_async_copy(
                                    dst_hbm.at[s, pl.ds(0, 4)],
                                    dst_v.at[gn], idsems[0]).wait()
                            start_gather(src_v.at[gn, 0], ob)
                return carry

            lax.fori_loop(0, 20, outer, 0)
            wait_scatter(1)
            plsc.subcore_barrier()
            pltpu.sync_copy(slab.at[pl.ds(base, 640)],
                            v_plane.at[pl.ds(base, 640)])


def _sc_prop(u, src_t, dst_t):
    mesh = plsc.VectorSubcoreMesh(core_axis_name="c", subcore_axis_name="s",
                                  num_cores=2, num_subcores=16)
    return pl.kernel(
        _sc_prop_body,
        out_type=jax.ShapeDtypeStruct((2, NPAD, 128), _f32),
        mesh=mesh,
        scratch_types=[
            pltpu.VMEM((2, 4, 128), jnp.int32),
            pltpu.VMEM((2, 4, 128), jnp.int32),
            pltpu.VMEM((2, 128, 128), _f32),
            pltpu.VMEM_SHARED((NPAD, 128), _f32),
        ] + [pltpu.SemaphoreType.DMA] * 8,
    )(u, src_t, dst_t)


def _row_mask(n, bn):
    rows = n * bn + lax.broadcasted_iota(jnp.int32, (bn, 128), 0)
    return jnp.where(rows < NN, 1.0, 0.0).astype(_f32)


def _acc_stats(stats_ref, z, n, bn):
    m = _row_mask(n, bn)
    zm = z * m
    s1 = jnp.sum(zm, axis=0, keepdims=True)
    s2 = jnp.sum(zm * z, axis=0, keepdims=True)
    blk = jnp.concatenate([s1, s2, jnp.zeros((6, z.shape[1]), _f32)], axis=0)

    @pl.when(n == 0)
    def _():
        stats_ref[...] = blk

    @pl.when(n != 0)
    def _():
        stats_ref[...] = stats_ref[...] + blk


def _dot(a, b):
    return jax.lax.dot_general(a, b, (((1,), (0,)), ((), ())),
                               preferred_element_type=_f32)


def _dis_body(deg_ref, d_ref):
    deg = deg_ref[...]
    d_ref[...] = jnp.where(deg > 0.0, jax.lax.rsqrt(jnp.maximum(deg, 1e-12)),
                           0.0)


def _dis_kernel(degplane):
    return pl.pallas_call(
        _dis_body,
        out_shape=jax.ShapeDtypeStruct((NPAD, 128), _f32),
        grid=(NBLK,),
        in_specs=[pl.BlockSpec((BN, 128), lambda n: (n, 0))],
        out_specs=pl.BlockSpec((BN, 128), lambda n: (n, 0)),
    )(degplane)


def _mm_negd_body(x_ref, w_ref, d_ref, o_ref):
    o = _dot(x_ref[...], w_ref[...])
    o_ref[0] = -d_ref[...] * o


def _mm_negd(x, w, d):
    return pl.pallas_call(
        _mm_negd_body,
        out_shape=jax.ShapeDtypeStruct((2, NPAD, 128), _f32),
        grid=(2, NBLK),
        in_specs=[
            pl.BlockSpec((BN, x.shape[1]), lambda c, n: (n, 0)),
            pl.BlockSpec((x.shape[1], 128), lambda c, n: (0, c)),
            pl.BlockSpec((BN, 128), lambda c, n: (n, 0)),
        ],
        out_specs=pl.BlockSpec((1, BN, 128), lambda c, n: (c, n, 0)),
    )(x, w, d)


def _a2_body(t0_ref, v_ref, d_ref, b_ref, z_ref, st_ref):
    n = pl.program_id(1)
    z = t0_ref[...] + d_ref[...] * v_ref[0] + b_ref[...]
    z_ref[...] = z
    _acc_stats(st_ref, z, n, BN)


def _conv_a2(t0, v, d, b):
    oc = t0.shape[1]
    return pl.pallas_call(
        _a2_body,
        out_shape=[jax.ShapeDtypeStruct((NPAD, oc), _f32),
                   jax.ShapeDtypeStruct((8, oc), _f32)],
        grid=(oc // 128, NBLK),
        in_specs=[
            pl.BlockSpec((BN, 128), lambda c, n: (n, c)),
            pl.BlockSpec((1, BN, 128), lambda c, n: (c, n, 0)),
            pl.BlockSpec((BN, 128), lambda c, n: (n, 0)),
            pl.BlockSpec((1, 128), lambda c, n: (0, c)),
        ],
        out_specs=[pl.BlockSpec((BN, 128), lambda c, n: (n, c)),
                   pl.BlockSpec((8, 128), lambda c, n: (0, c))],
    )(t0, v, d, b)


def _b2_body(t0_ref, v1_ref, v2_ref, d_ref, w1_ref, w2_ref, b_ref,
             z_ref, st_ref):
    n = pl.program_id(1)
    d = d_ref[...]
    t1 = jnp.concatenate([d * v1_ref[0], d * v1_ref[1]], axis=1)
    t2 = jnp.concatenate([d * v2_ref[0], d * v2_ref[1]], axis=1)
    z = (t0_ref[...] + _dot(t1, w1_ref[...]) +
         _dot(t2, w2_ref[...]) + b_ref[...])
    z_ref[...] = z
    _acc_stats(st_ref, z, n, BN)


def _conv_b2(t0, v1, v2, d, w1, w2t, b):
    ic = 256
    oc = t0.shape[1]
    return pl.pallas_call(
        _b2_body,
        out_shape=[jax.ShapeDtypeStruct((NPAD, oc), _f32),
                   jax.ShapeDtypeStruct((8, oc), _f32)],
        grid=(oc // 128, NBLK),
        in_specs=[
            pl.BlockSpec((BN, 128), lambda c, n: (n, c)),
            pl.BlockSpec((2, BN, 128), lambda c, n: (0, n, 0)),
            pl.BlockSpec((2, BN, 128), lambda c, n: (0, n, 0)),
            pl.BlockSpec((BN, 128), lambda c, n: (n, 0)),
            pl.BlockSpec((ic, 128), lambda c, n: (0, c)),
            pl.BlockSpec((ic, 128), lambda c, n: (0, c)),
            pl.BlockSpec((1, 128), lambda c, n: (0, c)),
        ],
        out_specs=[pl.BlockSpec((BN, 128), lambda c, n: (n, c)),
                   pl.BlockSpec((8, 128), lambda c, n: (0, c))],
    )(t0, v1, v2, d, w1, w2t, b)


def _mm_body(x_ref, w_ref, y_ref):
    y_ref[...] = _dot(x_ref[...], w_ref[...])


def _mm_plain(x, w):
    oc = w.shape[1]
    return pl.pallas_call(
        _mm_body,
        out_shape=jax.ShapeDtypeStruct((NPAD, oc), _f32),
        grid=(oc // 128, NBLK),
        in_specs=[
            pl.BlockSpec((BN, x.shape[1]), lambda c, n: (n, 0)),
            pl.BlockSpec((x.shape[1], 128), lambda c, n: (0, c)),
        ],
        out_specs=pl.BlockSpec((BN, 128), lambda c, n: (n, c)),
    )(x, w)


def _c1b_body(y_ref, d_ref, o_ref):
    o_ref[0] = -d_ref[...] * y_ref[...]


def _scale_negd_cols(y, d, col0):
    cb = col0 // 128
    return pl.pallas_call(
        _c1b_body,
        out_shape=jax.ShapeDtypeStruct((2, NPAD, 128), _f32),
        grid=(2, NBLK),
        in_specs=[
            pl.BlockSpec((BN, 128), lambda c, n: (n, c + cb)),
            pl.BlockSpec((BN, 128), lambda c, n: (n, 0)),
        ],
        out_specs=pl.BlockSpec((1, BN, 128), lambda c, n: (c, n, 0)),
    )(y, d)


def _b1_body(v_ref, d_ref, o_ref):
    d = d_ref[...]
    o_ref[0] = -d * d * v_ref[0]


def _scale_negd2(v, d):
    return pl.pallas_call(
        _b1_body,
        out_shape=jax.ShapeDtypeStruct((2, NPAD, 128), _f32),
        grid=(2, NBLK),
        in_specs=[
            pl.BlockSpec((1, BN, 128), lambda c, n: (c, n, 0)),
            pl.BlockSpec((BN, 128), lambda c, n: (n, 0)),
        ],
        out_specs=pl.BlockSpec((1, BN, 128), lambda c, n: (c, n, 0)),
    )(v, d)


def _c2_body(y_ref, v_ref, d_ref, o_ref):
    d = d_ref[...]
    o_ref[0] = -d * y_ref[...] - 2.0 * d * d * v_ref[0]


def _scale_c2(y, v, d, col0):
    cb = col0 // 128
    return pl.pallas_call(
        _c2_body,
        out_shape=jax.ShapeDtypeStruct((2, NPAD, 128), _f32),
        grid=(2, NBLK),
        in_specs=[
            pl.BlockSpec((BN, 128), lambda c, n: (n, c + cb)),
            pl.BlockSpec((1, BN, 128), lambda c, n: (c, n, 0)),
            pl.BlockSpec((BN, 128), lambda c, n: (n, 0)),
        ],
        out_specs=pl.BlockSpec((1, BN, 128), lambda c, n: (c, n, 0)),
    )(y, v, d)


def _c3_body(y0_ref, y2_ref, v_ref, d_ref, b_ref, z_ref, st_ref):
    n = pl.program_id(1)
    z = y0_ref[...] - y2_ref[...] + d_ref[...] * v_ref[0] + b_ref[...]
    z_ref[...] = z
    _acc_stats(st_ref, z, n, BN)


def _conv_c3(y, v, d, b):
    oc = 256
    return pl.pallas_call(
        _c3_body,
        out_shape=[jax.ShapeDtypeStruct((NPAD, oc), _f32),
                   jax.ShapeDtypeStruct((8, oc), _f32)],
        grid=(oc // 128, NBLK),
        in_specs=[
            pl.BlockSpec((BN, 128), lambda c, n: (n, c)),
            pl.BlockSpec((BN, 128), lambda c, n: (n, c + 4)),
            pl.BlockSpec((1, BN, 128), lambda c, n: (c, n, 0)),
            pl.BlockSpec((BN, 128), lambda c, n: (n, 0)),
            pl.BlockSpec((1, 128), lambda c, n: (0, c)),
        ],
        out_specs=[pl.BlockSpec((BN, 128), lambda c, n: (n, c)),
                   pl.BlockSpec((8, 128), lambda c, n: (0, c))],
    )(y, y, v, d, b)


def _d1_body(t0_ref, v_ref, d_ref, w1_ref, b_ref, r_ref, h_ref):
    d = d_ref[...]
    t = jnp.concatenate([d * v_ref[0], d * v_ref[1]], axis=1)
    z = (t0_ref[...] + _dot(t, w1_ref[...]) + b_ref[...] + r_ref[...])
    h_ref[...] = jnp.maximum(z, 0.0)


def _conv_d1(t0, v, d, w1, b, res):
    ic = 256
    oc = t0.shape[1]
    return pl.pallas_call(
        _d1_body,
        out_shape=jax.ShapeDtypeStruct((NPAD, oc), _f32),
        grid=(oc // 128, NBLK),
        in_specs=[
            pl.BlockSpec((BN, 128), lambda c, n: (n, c)),
            pl.BlockSpec((2, BN, 128), lambda c, n: (0, n, 0)),
            pl.BlockSpec((BN, 128), lambda c, n: (n, 0)),
            pl.BlockSpec((ic, 128), lambda c, n: (0, c)),
            pl.BlockSpec((1, 128), lambda c, n: (0, c)),
            pl.BlockSpec((BN, 128), lambda c, n: (n, c)),
        ],
        out_specs=pl.BlockSpec((BN, 128), lambda c, n: (n, c)),
    )(t0, v, d, w1, b, res)


def _norm_body_u(z_ref, st_ref, w_ref, b_ref, ms_ref, d_ref, h_ref, u_ref):
    _norm_core(z_ref, st_ref, w_ref, b_ref, ms_ref, h_ref, d_ref, u_ref)


def _norm_body(z_ref, st_ref, w_ref, b_ref, ms_ref, h_ref):
    _norm_core(z_ref, st_ref, w_ref, b_ref, ms_ref, h_ref, None, None)


def _norm_core(z_ref, st_ref, w_ref, b_ref, ms_ref, h_ref, d_ref, u_ref):
    z = z_ref[...]
    s1 = st_ref[0:1, :]
    s2 = st_ref[1:2, :]
    inv_n = 1.0 / float(NN)
    mean = s1 * inv_n
    mms = mean * ms_ref[...]
    var = s2 * inv_n - 2.0 * mms * mean + mms * mms
    out = z - mms
    h = w_ref[...] * out * jax.lax.rsqrt(var + 1e-5) + b_ref[...]
    h = jnp.where(h > 0.0, h, 0.2 * h)
    h_ref[...] = h
    if u_ref is not None:
        u_ref[0] = -d_ref[...] * h


def _norm_act(z, stats, w, b, ms):
    oc = z.shape[1]
    return pl.pallas_call(
        _norm_body,
        out_shape=jax.ShapeDtypeStruct((NPAD, oc), _f32),
        grid=(oc // 128, NBLK),
        in_specs=[
            pl.BlockSpec((BN, 128), lambda c, n: (n, c)),
            pl.BlockSpec((8, 128), lambda c, n: (0, c)),
            pl.BlockSpec((1, 128), lambda c, n: (0, c)),
            pl.BlockSpec((1, 128), lambda c, n: (0, c)),
            pl.BlockSpec((1, 128), lambda c, n: (0, c)),
        ],
        out_specs=pl.BlockSpec((BN, 128), lambda c, n: (n, c)),
    )(z, stats, w, b, ms)


def _norm_act_u(z, stats, w, b, ms, d):
    oc = z.shape[1]
    assert oc == 256
    return pl.pallas_call(
        _norm_body_u,
        out_shape=[jax.ShapeDtypeStruct((NPAD, oc), _f32),
                   jax.ShapeDtypeStruct((2, NPAD, 128), _f32)],
        grid=(oc // 128, NBLK),
        in_specs=[
            pl.BlockSpec((BN, 128), lambda c, n: (n, c)),
            pl.BlockSpec((8, 128), lambda c, n: (0, c)),
            pl.BlockSpec((1, 128), lambda c, n: (0, c)),
            pl.BlockSpec((1, 128), lambda c, n: (0, c)),
            pl.BlockSpec((1, 128), lambda c, n: (0, c)),
            pl.BlockSpec((BN, 128), lambda c, n: (n, 0)),
        ],
        out_specs=[pl.BlockSpec((BN, 128), lambda c, n: (n, c)),
                   pl.BlockSpec((1, BN, 128), lambda c, n: (c, n, 0))],
    )(z, stats, w, b, ms, d)


def _pool_body(h_ref, b_ref, s_ref, c_ref):
    n = pl.program_id(0)
    onehot = (b_ref[...] == lax.broadcasted_iota(jnp.int32, (1, NG), 1))
    onehot = onehot.astype(_f32)
    sums = jax.lax.dot_general(onehot, h_ref[...], (((0,), (0,)), ((), ())),
                               preferred_element_type=_f32)
    ones = jnp.ones((BN, 128), _f32)
    cnt = jax.lax.dot_general(onehot, ones, (((0,), (0,)), ((), ())),
                              preferred_element_type=_f32)

    @pl.when(n == 0)
    def _():
        s_ref[...] = sums
        c_ref[...] = cnt

    @pl.when(n != 0)
    def _():
        s_ref[...] = s_ref[...] + sums
        c_ref[...] = c_ref[...] + cnt


def _pool(h, batch2d):
    return pl.pallas_call(
        _pool_body,
        out_shape=[jax.ShapeDtypeStruct((NG, 512), _f32),
                   jax.ShapeDtypeStruct((NG, 128), _f32)],
        grid=(NBLK,),
        in_specs=[
            pl.BlockSpec((BN, 512), lambda n: (n, 0)),
            pl.BlockSpec((BN, 1), lambda n: (n, 0)),
        ],
        out_specs=[pl.BlockSpec((NG, 512), lambda n: (0, 0)),
                   pl.BlockSpec((NG, 128), lambda n: (0, 0))],
    )(h, batch2d)


def _head_body(s_ref, c_ref, w1_ref, b1_ref, w2_ref, b2_ref, o_ref):
    cnt = jnp.maximum(c_ref[:, 0:1], 1.0)
    pooled = s_ref[...] / cnt
    t = jnp.tanh(_dot(pooled, w1_ref[...]) + b1_ref[...])
    o_ref[...] = _dot(t, w2_ref[...]) + b2_ref[...]


def _head(sums, cnt, w1, b1, w2, b2):
    return pl.pallas_call(
        _head_body,
        out_shape=jax.ShapeDtypeStruct((NG, 12), _f32),
    )(sums, cnt, w1, b1, w2, b2)


def _prep_edges(edge_index):
    src = edge_index[0]
    dst = edge_index[1]
    pad = EPAD - EE
    src_p = jnp.concatenate([src, jnp.full((pad,), NN, jnp.int32)])
    dst_p = jnp.concatenate([dst, jnp.full((pad,), NN, jnp.int32)])
    return src_p.reshape(16, 80, 128), dst_p.reshape(16, 80, 128)


def _planes(a):
    return jnp.moveaxis(a.reshape(NPAD, 2, 128), 1, 0)


@jax.jit
def kernel(x, edge_index, edge_weight, batch, params):
    del edge_weight
    src_t, dst_t = _prep_edges(edge_index)

    xp = jnp.pad(x, ((0, NPAD - NN), (0, 0)))
    batch_p = jnp.pad(batch, (0, NPAD - NN), constant_values=NG)
    batch2d = batch_p.reshape(NPAD, 1)

    ones_planes = jnp.ones((2, NPAD, 128), _f32)
    degp = _sc_prop(ones_planes, dst_t, src_t)
    d = _dis_kernel(degp[0])

    def bias2d(bv):
        return bv.reshape(1, -1)

    h = xp
    for blk in range(4):
        i0 = blk * 4
        pA = params['conv%d' % (i0 + 1)]
        pB = params['conv%d' % (i0 + 2)]
        pC = params['conv%d' % (i0 + 3)]
        pD = params['conv%d' % (i0 + 4)]
        gA = params['bn%d' % (i0 + 1)]
        gB = params['bn%d' % (i0 + 2)]
        gC = params['bn%d' % (i0 + 3)]

        u = _mm_negd(h, pA['Ws'][1], d)
        t0 = _mm_plain(h, pA['Ws'][0])
        v = _sc_prop(u, src_t, dst_t)
        z, st = _conv_a2(t0, v, d, bias2d(pA['b']))
        h1, u0 = _norm_act_u(z, st, bias2d(gA['weight']), bias2d(gA['bias']),
                             bias2d(gA['mean_scale']), d)

        w0m = pB['Ws'][0] - pB['Ws'][2]
        w2t = 2.0 * pB['Ws'][2]
        t0 = _mm_plain(h1, w0m)
        v1 = _sc_prop(u0, src_t, dst_t)
        u1 = _scale_negd2(v1, d)
        v2 = _sc_prop(u1, src_t, dst_t)
        z, st = _conv_b2(t0, v1, v2, d, pB['Ws'][1], w2t, bias2d(pB['b']))
        h2 = _norm_act(z, st, bias2d(gB['weight']), bias2d(gB['bias']),
                       bias2d(gB['mean_scale']))

        wcat = jnp.concatenate([pC['Ws'][0], pC['Ws'][1], pC['Ws'][2]], axis=1)
        y = _mm_plain(h2, wcat)
        u2 = _scale_negd_cols(y, d, 512)
        vA = _sc_prop(u2, src_t, dst_t)
        um = _scale_c2(y, vA, d, 256)
        vB = _sc_prop(um, src_t, dst_t)
        z, st = _conv_c3(y, vB, d, bias2d(pC['b']))
        h3, u3 = _norm_act_u(z, st, bias2d(gC['weight']), bias2d(gC['bias']),
                             bias2d(gC['mean_scale']), d)

        t0 = _mm_plain(h3, pD['Ws'][0])
        v = _sc_prop(u3, src_t, dst_t)
        h = _conv_d1(t0, v, d, pD['Ws'][1], bias2d(pD['b']), xp)

    sums, cnt = _pool(h, batch2d)
    return _head(sums, cnt, params['linear1']['W'],
                 bias2d(params['linear1']['b']), params['linear2']['W'],
                 bias2d(params['linear2']['b']))
